# baseline (speedup 1.0000x reference)
import jax
import jax.numpy as jnp
from jax.experimental import pallas as pl
from jax.experimental.pallas import tpu as pltpu

_R = 512
_E_TOK = 256
_H = 512
_V = 1024
_VF = _V + 2
_MSG_LEN = 16
_TAU = 1.0
_UNIFORM_W = 0.1
_SOS_IDX = 0
_EOS_IDX = 1
_EPS = 1e-10
_VFP = 1152

_BB = 256


def _bcast(b_ref):
    return pltpu.repeat(b_ref[...], _BB // 128, axis=1)


def _body(pp_ref, u_ref, Wi_ref, bi_ref, Wih_ref, bih_ref,
          Whh_ref, bhh_ref, Wout_ref, bout_ref, ET_ref,
          out_ref, h_ref, x_ref):
    t = pl.program_id(1)
    row = jax.lax.broadcasted_iota(jnp.int32, (_VF, _BB), 0)

    @pl.when(t == 0)
    def _init():
        h_ref[...] = jnp.dot(Wi_ref[...], pp_ref[...],
                             preferred_element_type=jnp.float32) + _bcast(bi_ref)
        sos = (row == _SOS_IDX).astype(jnp.float32)
        x_ref[...] = jnp.dot(ET_ref[...], sos,
                             preferred_element_type=jnp.float32)
        out_ref[:, 0, :] = sos

    @pl.when((t >= 1) & (t <= _MSG_LEN - 2))
    def _step():
        h = h_ref[...]
        x = x_ref[...]
        gi = jnp.dot(Wih_ref[...], x,
                     preferred_element_type=jnp.float32) + _bcast(bih_ref)
        gh = jnp.dot(Whh_ref[...], h,
                     preferred_element_type=jnp.float32) + _bcast(bhh_ref)
        r = jax.nn.sigmoid(gi[:_H] + gh[:_H])
        z = jax.nn.sigmoid(gi[_H:2 * _H] + gh[_H:2 * _H])
        n = jnp.tanh(gi[2 * _H:] + r * gh[2 * _H:])
        h_new = (1.0 - z) * n + z * h
        h_ref[...] = h_new

        logits = jnp.dot(Wout_ref[...], h_new,
                         preferred_element_type=jnp.float32)[:_VF] + bout_ref[...]
        u = u_ref[0]
        g = -jnp.log(-jnp.log(u + _EPS) + _EPS)
        s = (logits + g) / _TAU
        m = jnp.max(s, axis=0, keepdims=True)
        p = jnp.exp(s - m)
        y = p / jnp.sum(p, axis=0, keepdims=True)
        y = (1.0 - _UNIFORM_W) * y + _UNIFORM_W / _V
        idx = jnp.argmax(y, axis=0, keepdims=True)
        onehot = (row == idx).astype(jnp.float32)
        st = onehot - y + y
        x_ref[...] = jnp.dot(ET_ref[...], st,
                             preferred_element_type=jnp.float32)
        for tt in range(1, _MSG_LEN - 1):
            @pl.when(t == tt)
            def _store(tt=tt, st=st):
                out_ref[:, tt % 8, :] = st

    @pl.when(t == _MSG_LEN - 1)
    def _eos():
        out_ref[:, 7, :] = (row == _EOS_IDX).astype(jnp.float32)


def kernel(proto0, proto1, gumbel_u, W_init, b_init, W_ih, b_ih, W_hh, b_hh,
           W_out, b_out, E):
    B = proto0.shape[0]
    nb = B // _BB
    grid = (nb, _MSG_LEN)

    ppT = jnp.concatenate([proto0, proto1], axis=1).T
    guT = jnp.swapaxes(gumbel_u, 1, 2)
    ET = E.T
    col = lambda v: jnp.broadcast_to(v[:, None], (v.shape[0], 128))

    full = lambda shape: pl.BlockSpec(shape, lambda b, t: (0,) * len(shape))
    out = pl.pallas_call(
        _body,
        grid=grid,
        in_specs=[
            pl.BlockSpec((2 * _R, _BB), lambda b, t: (0, b)),
            pl.BlockSpec((1, _VF, _BB),
                         lambda b, t: (jnp.clip(t - 1, 0, _MSG_LEN - 3), 0, b)),
            full((_H, 2 * _R)),
            full((_H, 128)),
            full((3 * _H, _E_TOK)),
            full((3 * _H, 128)),
            full((3 * _H, _H)),
            full((3 * _H, 128)),
            full((_VFP, _H)),
            full((_VF, _BB)),
            full((_E_TOK, _VF)),
        ],
        out_specs=pl.BlockSpec((_VF, 8, _BB), lambda b, t: (0, t // 8, b)),
        out_shape=jax.ShapeDtypeStruct((_VF, _MSG_LEN, B), jnp.float32),
        scratch_shapes=[
            pltpu.VMEM((_H, _BB), jnp.float32),
            pltpu.VMEM((_E_TOK, _BB), jnp.float32),
        ],
        compiler_params=pltpu.CompilerParams(
            dimension_semantics=("parallel", "arbitrary"),
            vmem_limit_bytes=60 * 1024 * 1024,
        ),
        name="sender_gru_lm",
    )(ppT, guT, W_init, col(b_init), W_ih, col(b_ih), W_hh, col(b_hh),
      jnp.pad(W_out, ((0, _VFP - _VF), (0, 0))),
      jnp.broadcast_to(b_out[:, None], (_VF, _BB)), ET)
    return jnp.transpose(out, (2, 1, 0))


# speedup vs baseline: 1.2684x; 1.2684x over previous
import jax
import jax.numpy as jnp
from jax.experimental import pallas as pl
from jax.experimental.pallas import tpu as pltpu

_R = 512
_E_TOK = 256
_H = 512
_V = 1024
_VF = _V + 2
_MSG_LEN = 16
_TAU = 1.0
_UNIFORM_W = 0.1
_SOS_IDX = 0
_EOS_IDX = 1
_EPS = 1e-10
_VFP = 1152

_BB = 512


def _bcast(b_ref):
    return pltpu.repeat(b_ref[...], _BB // 128, axis=1)


def _body(pp_ref, u_ref, Wi_ref, bi_ref, Wih_ref, bih_ref,
          Whh_ref, bhh_ref, Wout_ref, bout_ref, ET_ref,
          out_ref, h_ref, x_ref):
    t = pl.program_id(1)
    row = jax.lax.broadcasted_iota(jnp.int32, (_VF, _BB), 0)

    @pl.when(t == 0)
    def _init():
        h_ref[...] = jnp.dot(Wi_ref[...], pp_ref[...],
                             preferred_element_type=jnp.float32) + _bcast(bi_ref)
        sos = (row == _SOS_IDX).astype(jnp.float32)
        x_ref[...] = jnp.dot(ET_ref[...], sos,
                             preferred_element_type=jnp.float32)
        out_ref[:, 0, :] = sos

    @pl.when((t >= 1) & (t <= _MSG_LEN - 2))
    def _step():
        h = h_ref[...]
        x = x_ref[...]
        gi = jnp.dot(Wih_ref[...], x,
                     preferred_element_type=jnp.float32) + _bcast(bih_ref)
        gh = jnp.dot(Whh_ref[...], h,
                     preferred_element_type=jnp.float32) + _bcast(bhh_ref)
        r = jax.nn.sigmoid(gi[:_H] + gh[:_H])
        z = jax.nn.sigmoid(gi[_H:2 * _H] + gh[_H:2 * _H])
        n = jnp.tanh(gi[2 * _H:] + r * gh[2 * _H:])
        h_new = (1.0 - z) * n + z * h
        h_ref[...] = h_new

        logits = jnp.dot(Wout_ref[...], h_new,
                         preferred_element_type=jnp.float32)[:_VF] + _bcast(bout_ref)
        u = u_ref[0]
        g = -jnp.log(-jnp.log(u + _EPS) + _EPS)
        s = (logits + g) / _TAU
        m = jnp.max(s, axis=0, keepdims=True)
        p = jnp.exp(s - m)
        y = p / jnp.sum(p, axis=0, keepdims=True)
        y = (1.0 - _UNIFORM_W) * y + _UNIFORM_W / _V
        idx = jnp.argmax(y, axis=0, keepdims=True)
        onehot = (row == idx).astype(jnp.float32)
        st = onehot - y + y
        x_ref[...] = jnp.dot(ET_ref[...], st,
                             preferred_element_type=jnp.float32)
        for tt in range(1, _MSG_LEN - 1):
            @pl.when(t == tt)
            def _store(tt=tt, st=st):
                out_ref[:, tt % 8, :] = st

    @pl.when(t == _MSG_LEN - 1)
    def _eos():
        out_ref[:, 7, :] = (row == _EOS_IDX).astype(jnp.float32)


def kernel(proto0, proto1, gumbel_u, W_init, b_init, W_ih, b_ih, W_hh, b_hh,
           W_out, b_out, E):
    B = proto0.shape[0]
    nb = B // _BB
    grid = (nb, _MSG_LEN)

    ppT = jnp.concatenate([proto0, proto1], axis=1).T
    guT = jnp.swapaxes(gumbel_u, 1, 2)
    ET = E.T
    col = lambda v: jnp.broadcast_to(v[:, None], (v.shape[0], 128))

    full = lambda shape: pl.BlockSpec(shape, lambda b, t: (0,) * len(shape))
    out = pl.pallas_call(
        _body,
        grid=grid,
        in_specs=[
            pl.BlockSpec((2 * _R, _BB), lambda b, t: (0, b)),
            pl.BlockSpec((1, _VF, _BB),
                         lambda b, t: (jnp.clip(t - 1, 0, _MSG_LEN - 3), 0, b)),
            full((_H, 2 * _R)),
            full((_H, 128)),
            full((3 * _H, _E_TOK)),
            full((3 * _H, 128)),
            full((3 * _H, _H)),
            full((3 * _H, 128)),
            full((_VFP, _H)),
            full((_VF, 128)),
            full((_E_TOK, _VF)),
        ],
        out_specs=pl.BlockSpec((_VF, 8, _BB), lambda b, t: (0, t // 8, b)),
        out_shape=jax.ShapeDtypeStruct((_VF, _MSG_LEN, B), jnp.float32),
        scratch_shapes=[
            pltpu.VMEM((_H, _BB), jnp.float32),
            pltpu.VMEM((_E_TOK, _BB), jnp.float32),
        ],
        compiler_params=pltpu.CompilerParams(
            dimension_semantics=("parallel", "arbitrary"),
            vmem_limit_bytes=63 * 1024 * 1024,
        ),
        name="sender_gru_lm",
    )(ppT, guT, W_init, col(b_init), W_ih, col(b_ih), W_hh, col(b_hh),
      jnp.pad(W_out, ((0, _VFP - _VF), (0, 0))), col(b_out), ET)
    return jnp.transpose(out, (2, 1, 0))
